# baseline (speedup 1.0000x reference)
import jax
import jax.numpy as jnp
from jax import lax
from jax.experimental import pallas as pl
from jax.experimental.pallas import tpu as pltpu
from jax.experimental.pallas import tpu_sc as plsc

SIZE = 64
CAP = 1000000
NC, NS = 2, 16
NW = NC * NS
NFIN = NW * 16
CHUNK = 320
NCHUNK = 12
RPT = CHUNK * NCHUNK
ROWS_SC = NW * RPT
R_TC = CAP - ROWS_SC
TCB = 1024
NBLK = -(-R_TC // TCB)
GROUPS = CHUNK // 16
_SC_PARAMS = dict(compiler_params=pltpu.CompilerParams(needs_layout_passes=False))


def _proj_body(q_ref, w_ref, b_ref, o_ref):
    qb = q_ref[...].astype(jnp.bfloat16)
    wb = w_ref[...].astype(jnp.bfloat16)
    pq = lax.dot_general(qb, wb, (((1,), (1,)), ((), ())),
                         preferred_element_type=jnp.float32) + b_ref[...]
    nrm = jnp.maximum(jnp.sqrt(jnp.sum(pq * pq)), 1e-8)
    pqn = pq / nrm
    o_ref[...] = pqn.astype(jnp.bfloat16).astype(jnp.float32)


def _project(query, W, b):
    return pl.pallas_call(
        _proj_body,
        out_shape=jax.ShapeDtypeStruct((1, SIZE), jnp.float32),
    )(query, W, b.reshape(1, SIZE))


def _tcscan_body(pqb_ref, keys_ref, conf_ref, idx_ref, rmax, ridx):
    i = pl.program_id(0)

    @pl.when(i == 0)
    def _():
        rmax[0] = jnp.float32(-jnp.inf)
        ridx[0] = jnp.int32(0)

    x = keys_ref[...]
    ss = jnp.sum(x * x, axis=-1, keepdims=True)
    normc = jnp.maximum(jnp.sqrt(ss), 1e-8)
    xnb = (x / normc).astype(jnp.bfloat16).astype(jnp.float32)
    qb = pqb_ref[...]
    sims = lax.dot_general(xnb, qb, (((1,), (1,)), ((), ())),
                           preferred_element_type=jnp.float32)
    mx = jnp.max(sims)
    iota = lax.broadcasted_iota(jnp.int32, (TCB, 1), 0)
    am = jnp.min(jnp.where(sims == mx, iota, jnp.int32(0x7FFFFFFF)))
    gidx = i * TCB + am
    pred = mx > rmax[0]
    rmax[0] = jnp.where(pred, mx, rmax[0])
    ridx[0] = jnp.where(pred, gidx, ridx[0])
    conf_ref[...] = jnp.full((1, 128), rmax[0], jnp.float32)
    idx_ref[...] = jnp.full((1, 128), ridx[0], jnp.int32)


def _tcscan(pqn, cache_keys):
    return pl.pallas_call(
        _tcscan_body,
        grid=(NBLK,),
        in_specs=[
            pl.BlockSpec((1, SIZE), lambda i: (0, 0)),
            pl.BlockSpec((TCB, SIZE), lambda i: (i, 0)),
        ],
        out_specs=[
            pl.BlockSpec((1, 128), lambda i: (0, 0)),
            pl.BlockSpec((1, 128), lambda i: (0, 0)),
        ],
        out_shape=(jax.ShapeDtypeStruct((1, 128), jnp.float32),
                   jax.ShapeDtypeStruct((1, 128), jnp.int32)),
        scratch_shapes=[pltpu.SMEM((1,), jnp.float32),
                        pltpu.SMEM((1,), jnp.int32)],
    )(pqn, cache_keys)


def _scan_body(pqb_hbm, keys_hbm, sims_out, idx_out,
               pqb_v, buf0, buf1, kbuf, stage_s, stage_i,
               sem0, sem1, semg):
    cid = lax.axis_index("c")
    sid = lax.axis_index("s")
    wid = sid * NC + cid
    base_row = R_TC + wid * RPT

    pltpu.sync_copy(pqb_hbm, pqb_v)
    pqvecs = [pqb_v[pl.ds(k * 16, 16)] for k in range(SIZE // 16)]
    pq = [pqvecs[d // 16][d % 16] for d in range(SIZE)]
    riota = lax.iota(jnp.int32, 16)

    def start(cidx, buf, sem):
        off = base_row + cidx * CHUNK
        pltpu.async_copy(keys_hbm.at[pl.ds(off, CHUNK), :], buf, sem)

    def wait(buf, sem):
        pltpu.make_async_copy(keys_hbm.at[pl.ds(0, CHUNK), :], buf, sem).wait()

    def process(buf, chunk_row_base, bk, bi):
        def gbody(gr, carry):
            bk, bi = carry
            rows = riota + gr * 16
            zero = gr * 0
            dot = jnp.zeros((16,), jnp.float32)
            ss = jnp.zeros((16,), jnp.float32)
            for d in range(SIZE):
                cols = jnp.full((16,), zero + d, jnp.int32)
                v = plsc.load_gather(buf, [rows, cols])
                dot = dot + v * pq[d]
                ss = ss + v * v
            key = dot * jnp.abs(dot) / jnp.maximum(ss, 1e-16)
            idxv = riota + (chunk_row_base + gr * 16)
            take = key > bk
            bk = jnp.where(take, key, bk)
            bi = jnp.where(take, idxv, bi)
            return bk, bi
        return lax.fori_loop(0, GROUPS, gbody, (bk, bi))

    start(0, buf0, sem0)
    start(1, buf1, sem1)
    bk0 = jnp.full((16,), -jnp.inf, jnp.float32)
    bi0 = jnp.zeros((16,), jnp.int32)

    def cbody(g, carry):
        bk, bi = carry
        for b, (buf, sem) in enumerate(((buf0, sem0), (buf1, sem1))):
            cidx = 2 * g + b
            wait(buf, sem)
            bk, bi = process(buf, base_row + cidx * CHUNK, bk, bi)

            @pl.when(cidx + 2 < NCHUNK)
            def _():
                start(cidx + 2, buf, sem)
        return bk, bi

    _, bi = lax.fori_loop(0, NCHUNK // 2, cbody, (bk0, bi0))

    stage_i[...] = bi
    for j in range(16):
        pltpu.async_copy(keys_hbm.at[pl.ds(bi[j], 1), :],
                         kbuf.at[pl.ds(j, 1), :], semg)
    for j in range(16):
        pltpu.make_async_copy(keys_hbm.at[pl.ds(0, 1), :],
                              kbuf.at[pl.ds(j, 1), :], semg).wait()

    ss = jnp.zeros((16,), jnp.float32)
    for d in range(SIZE):
        v = plsc.load_gather(kbuf, [riota, jnp.full((16,), d, jnp.int32)])
        ss = ss + v * v
    ssc = jnp.maximum(ss, 1e-30)
    yi = jnp.int32(0x5F3759DF) - lax.shift_right_logical(
        plsc.bitcast(ssc, jnp.int32), 1)
    y = plsc.bitcast(yi, jnp.float32)
    for _ in range(3):
        y = y * (1.5 - 0.5 * ssc * y * y)
    h = ssc * y
    h = 0.5 * (h + ssc / h)
    inv = 1.0 / jnp.maximum(h, 1e-8)
    acc = jnp.zeros((16,), jnp.float32)
    for d in range(SIZE):
        v = plsc.load_gather(kbuf, [riota, jnp.full((16,), d, jnp.int32)])
        t = v * inv
        tb = plsc.bitcast(t, jnp.int32)
        tb = tb + 0x7FFF + (lax.shift_right_logical(tb, 16) & 1)
        t = plsc.bitcast(tb & jnp.int32(-65536), jnp.float32)
        acc = acc + t * pq[d]

    stage_s[...] = acc
    pltpu.sync_copy(stage_s, sims_out.at[pl.ds(wid * 16, 16)])
    pltpu.sync_copy(stage_i, idx_out.at[pl.ds(wid * 16, 16)])


def _pick_body(sims_hbm, idx_hbm, tcc_hbm, tci_hbm, vals_hbm, conf_out, val_out,
               sbuf, ibuf, tcbuf, tibuf, cbuf, rowbuf, sem):
    cid = lax.axis_index("c")
    sid = lax.axis_index("s")
    wid = sid * NC + cid

    @pl.when(wid == 0)
    def _():
        pltpu.sync_copy(sims_hbm, sbuf)
        pltpu.sync_copy(idx_hbm, ibuf)
        pltpu.sync_copy(tcc_hbm, tcbuf)
        pltpu.sync_copy(tci_hbm, tibuf)
        bs = sbuf[pl.ds(0, 16)]
        bi = ibuf[pl.ds(0, 16)]
        for t in range(1, NW):
            sv = sbuf[pl.ds(t * 16, 16)]
            iv = ibuf[pl.ds(t * 16, 16)]
            take = (sv > bs) | ((sv == bs) & (iv < bi))
            bs = jnp.where(take, sv, bs)
            bi = jnp.where(take, iv, bi)
        mx = jnp.max(bs)
        cand = jnp.where(bs == mx, bi, jnp.int32(0x7FFFFFFF))
        bidx = jnp.min(cand)
        tcs = tcbuf[0, pl.ds(0, 16)]
        tci = tibuf[0, pl.ds(0, 16)]
        tcs0 = tcs[0]
        tci0 = tci[0]
        better = (tcs0 > mx) | ((tcs0 == mx) & (tci0 < bidx))
        mx = jnp.where(better, tcs0, mx)
        bidx = jnp.where(better, tci0, bidx)
        cbuf[...] = jnp.full((16,), mx, jnp.float32)
        pltpu.sync_copy(cbuf, conf_out)
        pltpu.async_copy(vals_hbm.at[pl.ds(bidx, 1), :], rowbuf, sem).wait()
        pltpu.sync_copy(rowbuf.at[0], val_out)


def _mesh():
    return plsc.VectorSubcoreMesh(core_axis_name="c", subcore_axis_name="s",
                                  num_cores=NC, num_subcores=NS)


def kernel(query, W, b, cache_keys, cache_values):
    pqn = _project(query, W, b)
    pqn_flat = pqn.reshape(SIZE)

    scan = pl.kernel(
        _scan_body,
        out_type=(jax.ShapeDtypeStruct((NFIN,), jnp.float32),
                  jax.ShapeDtypeStruct((NFIN,), jnp.int32)),
        mesh=_mesh(),
        scratch_types=[
            pltpu.VMEM((SIZE,), jnp.float32),
            pltpu.VMEM((CHUNK, SIZE), jnp.float32),
            pltpu.VMEM((CHUNK, SIZE), jnp.float32),
            pltpu.VMEM((16, SIZE), jnp.float32),
            pltpu.VMEM((16,), jnp.float32),
            pltpu.VMEM((16,), jnp.int32),
            pltpu.SemaphoreType.DMA,
            pltpu.SemaphoreType.DMA,
            pltpu.SemaphoreType.DMA,
        ],
        **_SC_PARAMS,
    )
    sims, fidx = scan(pqn_flat, cache_keys)
    conf_tc, idx_tc = _tcscan(pqn, cache_keys)

    pick = pl.kernel(
        _pick_body,
        out_type=(jax.ShapeDtypeStruct((16,), jnp.float32),
                  jax.ShapeDtypeStruct((SIZE,), jnp.float32)),
        mesh=_mesh(),
        scratch_types=[
            pltpu.VMEM((NFIN,), jnp.float32),
            pltpu.VMEM((NFIN,), jnp.int32),
            pltpu.VMEM((1, 128), jnp.float32),
            pltpu.VMEM((1, 128), jnp.int32),
            pltpu.VMEM((16,), jnp.float32),
            pltpu.VMEM((1, SIZE), jnp.float32),
            pltpu.SemaphoreType.DMA,
        ],
        **_SC_PARAMS,
    )
    conf, row = pick(sims, fidx, conf_tc, idx_tc, cache_values)
    return row.reshape(1, SIZE), conf[0]


# speedup vs baseline: 1.0144x; 1.0144x over previous
import jax
import jax.numpy as jnp
from jax import lax
from jax.experimental import pallas as pl
from jax.experimental.pallas import tpu as pltpu
from jax.experimental.pallas import tpu_sc as plsc

SIZE = 64
CAP = 1000000
NC, NS = 2, 16
NW = NC * NS
NFIN = NW * 16
CHUNK = 320
NCHUNK = 12
RPT = CHUNK * NCHUNK
ROWS_SC = NW * RPT
R_TC = CAP - ROWS_SC
TCB = 1024
NBLK = -(-R_TC // TCB)
GROUPS = CHUNK // 16
_SC_PARAMS = dict(compiler_params=pltpu.CompilerParams(needs_layout_passes=False))


def _proj_body(q_ref, w_ref, b_ref, o_ref):
    qb = q_ref[...].astype(jnp.bfloat16)
    wb = w_ref[...].astype(jnp.bfloat16)
    pq = lax.dot_general(qb, wb, (((1,), (1,)), ((), ())),
                         preferred_element_type=jnp.float32) + b_ref[...]
    nrm = jnp.maximum(jnp.sqrt(jnp.sum(pq * pq)), 1e-8)
    pqn = pq / nrm
    o_ref[...] = pqn.astype(jnp.bfloat16).astype(jnp.float32)


def _project(query, W, b):
    return pl.pallas_call(
        _proj_body,
        out_shape=jax.ShapeDtypeStruct((1, SIZE), jnp.float32),
    )(query, W, b.reshape(1, SIZE))


def _tcscan_body(pqb_ref, keys_ref, fin_ref):
    i = pl.program_id(0)

    @pl.when(i == 0)
    def _():
        fin_ref[...] = jnp.zeros((8, 128), jnp.int32)

    x = keys_ref[...]
    qb = pqb_ref[...]
    dot = lax.dot_general(x, qb, (((1,), (1,)), ((), ())),
                          preferred_element_type=jnp.float32)
    x2 = x * x
    ones = jnp.ones((1, SIZE), jnp.float32)
    ss = lax.dot_general(x2, ones, (((1,), (1,)), ((), ())),
                         preferred_element_type=jnp.float32)
    key = dot * jnp.abs(dot) / jnp.maximum(ss, 1e-16)
    mx = jnp.max(key)
    iota = lax.broadcasted_iota(jnp.int32, (TCB, 1), 0)
    am = jnp.min(jnp.where(key == mx, iota, jnp.int32(0x7FFFFFFF)))
    gidx = i * TCB + am
    mrow = lax.broadcasted_iota(jnp.int32, (8, 128), 0) == (i // 128)
    mcol = lax.broadcasted_iota(jnp.int32, (8, 128), 1) == (i % 128)
    fin_ref[...] = jnp.where(mrow & mcol, gidx, fin_ref[...])


def _tcscan(pqn, cache_keys):
    return pl.pallas_call(
        _tcscan_body,
        grid=(NBLK,),
        in_specs=[
            pl.BlockSpec((1, SIZE), lambda i: (0, 0)),
            pl.BlockSpec((TCB, SIZE), lambda i: (i, 0)),
        ],
        out_specs=pl.BlockSpec((8, 128), lambda i: (0, 0)),
        out_shape=jax.ShapeDtypeStruct((8, 128), jnp.int32),
    )(pqn, cache_keys)


NTC = 1024
TPW = NTC // NW


def _rescore_body(pqb_hbm, tcidx_hbm, keys_hbm, sims_out,
                  pqb_v, ibuf, kbuf, stage_a, stage_b, semg):
    cid = lax.axis_index("c")
    sid = lax.axis_index("s")
    wid = sid * NC + cid
    j0 = wid * TPW

    pltpu.sync_copy(pqb_hbm, pqb_v)
    pltpu.sync_copy(tcidx_hbm, ibuf)
    pqvecs = [pqb_v[pl.ds(k * 16, 16)] for k in range(SIZE // 16)]
    pq = [pqvecs[d // 16][d % 16] for d in range(SIZE)]
    riota = lax.iota(jnp.int32, 16)

    ivs = [ibuf[pl.ds(j0 + h * 16, 16)] for h in range(TPW // 16)]
    for h in range(TPW // 16):
        for j in range(16):
            pltpu.async_copy(keys_hbm.at[pl.ds(ivs[h][j], 1), :],
                             kbuf.at[pl.ds(h * 16 + j, 1), :], semg)
    for j in range(TPW):
        pltpu.make_async_copy(keys_hbm.at[pl.ds(0, 1), :],
                              kbuf.at[pl.ds(j, 1), :], semg).wait()

    for h, stage in enumerate((stage_a, stage_b)):
        rows = riota + h * 16
        ss = jnp.zeros((16,), jnp.float32)
        for d in range(SIZE):
            v = plsc.load_gather(kbuf, [rows, jnp.full((16,), d, jnp.int32)])
            ss = ss + v * v
        ssc = jnp.maximum(ss, 1e-30)
        yi = jnp.int32(0x5F3759DF) - lax.shift_right_logical(
            plsc.bitcast(ssc, jnp.int32), 1)
        y = plsc.bitcast(yi, jnp.float32)
        for _ in range(3):
            y = y * (1.5 - 0.5 * ssc * y * y)
        h2 = ssc * y
        h2 = 0.5 * (h2 + ssc / h2)
        inv = 1.0 / jnp.maximum(h2, 1e-8)
        acc = jnp.zeros((16,), jnp.float32)
        for d in range(SIZE):
            v = plsc.load_gather(kbuf, [rows, jnp.full((16,), d, jnp.int32)])
            t = v * inv
            tb = plsc.bitcast(t, jnp.int32)
            tb = tb + 0x7FFF + (lax.shift_right_logical(tb, 16) & 1)
            t = plsc.bitcast(tb & jnp.int32(-65536), jnp.float32)
            acc = acc + t * pq[d]
        stage[...] = acc
        pltpu.sync_copy(stage, sims_out.at[pl.ds(j0 + h * 16, 16)])


def _scan_body(pqb_hbm, keys_hbm, sims_out, idx_out,
               pqb_v, buf0, buf1, kbuf, stage_s, stage_i,
               sem0, sem1, semg):
    cid = lax.axis_index("c")
    sid = lax.axis_index("s")
    wid = sid * NC + cid
    base_row = R_TC + wid * RPT

    pltpu.sync_copy(pqb_hbm, pqb_v)
    pqvecs = [pqb_v[pl.ds(k * 16, 16)] for k in range(SIZE // 16)]
    pq = [pqvecs[d // 16][d % 16] for d in range(SIZE)]
    riota = lax.iota(jnp.int32, 16)

    def start(cidx, buf, sem):
        off = base_row + cidx * CHUNK
        pltpu.async_copy(keys_hbm.at[pl.ds(off, CHUNK), :], buf, sem)

    def wait(buf, sem):
        pltpu.make_async_copy(keys_hbm.at[pl.ds(0, CHUNK), :], buf, sem).wait()

    def process(buf, chunk_row_base, bk, bi):
        def gbody(gr, carry):
            bk, bi = carry
            rows = riota + gr * 16
            zero = gr * 0
            dot = jnp.zeros((16,), jnp.float32)
            ss = jnp.zeros((16,), jnp.float32)
            for d in range(SIZE):
                cols = jnp.full((16,), zero + d, jnp.int32)
                v = plsc.load_gather(buf, [rows, cols])
                dot = dot + v * pq[d]
                ss = ss + v * v
            key = dot * jnp.abs(dot) / jnp.maximum(ss, 1e-16)
            idxv = riota + (chunk_row_base + gr * 16)
            take = key > bk
            bk = jnp.where(take, key, bk)
            bi = jnp.where(take, idxv, bi)
            return bk, bi
        return lax.fori_loop(0, GROUPS, gbody, (bk, bi))

    start(0, buf0, sem0)
    start(1, buf1, sem1)
    bk0 = jnp.full((16,), -jnp.inf, jnp.float32)
    bi0 = jnp.zeros((16,), jnp.int32)

    def cbody(g, carry):
        bk, bi = carry
        for b, (buf, sem) in enumerate(((buf0, sem0), (buf1, sem1))):
            cidx = 2 * g + b
            wait(buf, sem)
            bk, bi = process(buf, base_row + cidx * CHUNK, bk, bi)

            @pl.when(cidx + 2 < NCHUNK)
            def _():
                start(cidx + 2, buf, sem)
        return bk, bi

    _, bi = lax.fori_loop(0, NCHUNK // 2, cbody, (bk0, bi0))

    stage_i[...] = bi
    for j in range(16):
        pltpu.async_copy(keys_hbm.at[pl.ds(bi[j], 1), :],
                         kbuf.at[pl.ds(j, 1), :], semg)
    for j in range(16):
        pltpu.make_async_copy(keys_hbm.at[pl.ds(0, 1), :],
                              kbuf.at[pl.ds(j, 1), :], semg).wait()

    ss = jnp.zeros((16,), jnp.float32)
    for d in range(SIZE):
        v = plsc.load_gather(kbuf, [riota, jnp.full((16,), d, jnp.int32)])
        ss = ss + v * v
    ssc = jnp.maximum(ss, 1e-30)
    yi = jnp.int32(0x5F3759DF) - lax.shift_right_logical(
        plsc.bitcast(ssc, jnp.int32), 1)
    y = plsc.bitcast(yi, jnp.float32)
    for _ in range(3):
        y = y * (1.5 - 0.5 * ssc * y * y)
    h = ssc * y
    h = 0.5 * (h + ssc / h)
    inv = 1.0 / jnp.maximum(h, 1e-8)
    acc = jnp.zeros((16,), jnp.float32)
    for d in range(SIZE):
        v = plsc.load_gather(kbuf, [riota, jnp.full((16,), d, jnp.int32)])
        t = v * inv
        tb = plsc.bitcast(t, jnp.int32)
        tb = tb + 0x7FFF + (lax.shift_right_logical(tb, 16) & 1)
        t = plsc.bitcast(tb & jnp.int32(-65536), jnp.float32)
        acc = acc + t * pq[d]

    stage_s[...] = acc
    pltpu.sync_copy(stage_s, sims_out.at[pl.ds(wid * 16, 16)])
    pltpu.sync_copy(stage_i, idx_out.at[pl.ds(wid * 16, 16)])


def _pick_body(sims_hbm, idx_hbm, tsims_hbm, tidx_hbm, vals_hbm,
               conf_out, val_out,
               sbuf, ibuf, tsbuf, tibuf, cbuf, rowbuf, sem):
    cid = lax.axis_index("c")
    sid = lax.axis_index("s")
    wid = sid * NC + cid

    @pl.when(wid == 0)
    def _():
        pltpu.sync_copy(sims_hbm, sbuf)
        pltpu.sync_copy(idx_hbm, ibuf)
        pltpu.sync_copy(tsims_hbm, tsbuf)
        pltpu.sync_copy(tidx_hbm, tibuf)
        bs = sbuf[pl.ds(0, 16)]
        bi = ibuf[pl.ds(0, 16)]
        for t in range(1, NW):
            sv = sbuf[pl.ds(t * 16, 16)]
            iv = ibuf[pl.ds(t * 16, 16)]
            take = (sv > bs) | ((sv == bs) & (iv < bi))
            bs = jnp.where(take, sv, bs)
            bi = jnp.where(take, iv, bi)
        for t in range(NTC // 16):
            sv = tsbuf[pl.ds(t * 16, 16)]
            iv = tibuf[pl.ds(t * 16, 16)]
            take = (sv > bs) | ((sv == bs) & (iv < bi))
            bs = jnp.where(take, sv, bs)
            bi = jnp.where(take, iv, bi)
        mx = jnp.max(bs)
        cand = jnp.where(bs == mx, bi, jnp.int32(0x7FFFFFFF))
        bidx = jnp.min(cand)
        cbuf[...] = jnp.full((16,), mx, jnp.float32)
        pltpu.sync_copy(cbuf, conf_out)
        pltpu.async_copy(vals_hbm.at[pl.ds(bidx, 1), :], rowbuf, sem).wait()
        pltpu.sync_copy(rowbuf.at[0], val_out)


def _mesh():
    return plsc.VectorSubcoreMesh(core_axis_name="c", subcore_axis_name="s",
                                  num_cores=NC, num_subcores=NS)


def kernel(query, W, b, cache_keys, cache_values):
    pqn = _project(query, W, b)
    pqn_flat = pqn.reshape(SIZE)

    scan = pl.kernel(
        _scan_body,
        out_type=(jax.ShapeDtypeStruct((NFIN,), jnp.float32),
                  jax.ShapeDtypeStruct((NFIN,), jnp.int32)),
        mesh=_mesh(),
        scratch_types=[
            pltpu.VMEM((SIZE,), jnp.float32),
            pltpu.VMEM((CHUNK, SIZE), jnp.float32),
            pltpu.VMEM((CHUNK, SIZE), jnp.float32),
            pltpu.VMEM((16, SIZE), jnp.float32),
            pltpu.VMEM((16,), jnp.float32),
            pltpu.VMEM((16,), jnp.int32),
            pltpu.SemaphoreType.DMA,
            pltpu.SemaphoreType.DMA,
            pltpu.SemaphoreType.DMA,
        ],
        **_SC_PARAMS,
    )
    sims, fidx = scan(pqn_flat, cache_keys)
    tcidx = _tcscan(pqn, cache_keys).reshape(NTC)

    rescore = pl.kernel(
        _rescore_body,
        out_type=jax.ShapeDtypeStruct((NTC,), jnp.float32),
        mesh=_mesh(),
        scratch_types=[
            pltpu.VMEM((SIZE,), jnp.float32),
            pltpu.VMEM((NTC,), jnp.int32),
            pltpu.VMEM((TPW, SIZE), jnp.float32),
            pltpu.VMEM((16,), jnp.float32),
            pltpu.VMEM((16,), jnp.float32),
            pltpu.SemaphoreType.DMA,
        ],
        **_SC_PARAMS,
    )
    tsims = rescore(pqn_flat, tcidx, cache_keys)

    pick = pl.kernel(
        _pick_body,
        out_type=(jax.ShapeDtypeStruct((16,), jnp.float32),
                  jax.ShapeDtypeStruct((SIZE,), jnp.float32)),
        mesh=_mesh(),
        scratch_types=[
            pltpu.VMEM((NFIN,), jnp.float32),
            pltpu.VMEM((NFIN,), jnp.int32),
            pltpu.VMEM((NTC,), jnp.float32),
            pltpu.VMEM((NTC,), jnp.int32),
            pltpu.VMEM((16,), jnp.float32),
            pltpu.VMEM((1, SIZE), jnp.float32),
            pltpu.SemaphoreType.DMA,
        ],
        **_SC_PARAMS,
    )
    conf, row = pick(sims, fidx, tsims, tcidx, cache_values)
    return row.reshape(1, SIZE), conf[0]


# speedup vs baseline: 1.2817x; 1.2635x over previous
import jax
import jax.numpy as jnp
from jax import lax
from jax.experimental import pallas as pl
from jax.experimental.pallas import tpu as pltpu
from jax.experimental.pallas import tpu_sc as plsc

SIZE = 64
CAP = 1000000
NC, NS = 2, 16
NW = NC * NS
NFIN = NW * 16
CHUNK = 320
NCHUNK = 12
RPT = CHUNK * NCHUNK
ROWS_SC = NW * RPT
R_TC = CAP - ROWS_SC
TCB = 4096
NBLK = -(-R_TC // TCB)
GROUPS = CHUNK // 16
_SC_PARAMS = dict(compiler_params=pltpu.CompilerParams(needs_layout_passes=False))


def _proj_body(q_ref, w_ref, b_ref, o_ref):
    qb = q_ref[...].astype(jnp.bfloat16)
    wb = w_ref[...].astype(jnp.bfloat16)
    pq = lax.dot_general(qb, wb, (((1,), (1,)), ((), ())),
                         preferred_element_type=jnp.float32) + b_ref[...]
    nrm = jnp.maximum(jnp.sqrt(jnp.sum(pq * pq)), 1e-8)
    pqn = pq / nrm
    o_ref[...] = pqn.astype(jnp.bfloat16).astype(jnp.float32)


def _project(query, W, b):
    return pl.pallas_call(
        _proj_body,
        out_shape=jax.ShapeDtypeStruct((1, SIZE), jnp.float32),
    )(query, W, b.reshape(1, SIZE))


def _tcscan_body(pqb_ref, keys_ref, fin_ref):
    i = pl.program_id(0)

    @pl.when(i == 0)
    def _():
        fin_ref[...] = jnp.zeros((8, 128), jnp.int32)

    x = keys_ref[...]
    qb = pqb_ref[...]
    dot = lax.dot_general(x, qb, (((1,), (1,)), ((), ())),
                          preferred_element_type=jnp.float32)
    x2 = x * x
    ones = jnp.ones((1, SIZE), jnp.float32)
    ss = lax.dot_general(x2, ones, (((1,), (1,)), ((), ())),
                         preferred_element_type=jnp.float32)
    key = dot * jnp.abs(dot) / jnp.maximum(ss, 1e-16)
    mx = jnp.max(key)
    iota = lax.broadcasted_iota(jnp.int32, (TCB, 1), 0)
    am = jnp.min(jnp.where(key == mx, iota, jnp.int32(0x7FFFFFFF)))
    gidx = i * TCB + am
    mrow = lax.broadcasted_iota(jnp.int32, (8, 128), 0) == (i // 128)
    mcol = lax.broadcasted_iota(jnp.int32, (8, 128), 1) == (i % 128)
    fin_ref[...] = jnp.where(mrow & mcol, gidx, fin_ref[...])


def _tcscan(pqn, cache_keys):
    return pl.pallas_call(
        _tcscan_body,
        grid=(NBLK,),
        in_specs=[
            pl.BlockSpec((1, SIZE), lambda i: (0, 0)),
            pl.BlockSpec((TCB, SIZE), lambda i: (i, 0)),
        ],
        out_specs=pl.BlockSpec((8, 128), lambda i: (0, 0)),
        out_shape=jax.ShapeDtypeStruct((8, 128), jnp.int32),
    )(pqn, cache_keys)


NTC = 1024
TPW = NTC // NW


def _rescore_body(pqb_hbm, tcidx_hbm, keys_hbm, sims_out,
                  pqb_v, ibuf, kbuf, stage_a, stage_b, semg):
    cid = lax.axis_index("c")
    sid = lax.axis_index("s")
    wid = sid * NC + cid
    j0 = wid * TPW

    pltpu.sync_copy(pqb_hbm, pqb_v)
    pltpu.sync_copy(tcidx_hbm, ibuf)
    pqvecs = [pqb_v[pl.ds(k * 16, 16)] for k in range(SIZE // 16)]
    pq = [pqvecs[d // 16][d % 16] for d in range(SIZE)]
    riota = lax.iota(jnp.int32, 16)

    ivs = [ibuf[pl.ds(j0 + h * 16, 16)] for h in range(TPW // 16)]
    for h in range(TPW // 16):
        for j in range(16):
            pltpu.async_copy(keys_hbm.at[pl.ds(ivs[h][j], 1), :],
                             kbuf.at[pl.ds(h * 16 + j, 1), :], semg)
    for j in range(TPW):
        pltpu.make_async_copy(keys_hbm.at[pl.ds(0, 1), :],
                              kbuf.at[pl.ds(j, 1), :], semg).wait()

    for h, stage in enumerate((stage_a, stage_b)):
        rows = riota + h * 16
        ss = jnp.zeros((16,), jnp.float32)
        for d in range(SIZE):
            v = plsc.load_gather(kbuf, [rows, jnp.full((16,), d, jnp.int32)])
            ss = ss + v * v
        ssc = jnp.maximum(ss, 1e-30)
        yi = jnp.int32(0x5F3759DF) - lax.shift_right_logical(
            plsc.bitcast(ssc, jnp.int32), 1)
        y = plsc.bitcast(yi, jnp.float32)
        for _ in range(3):
            y = y * (1.5 - 0.5 * ssc * y * y)
        h2 = ssc * y
        h2 = 0.5 * (h2 + ssc / h2)
        inv = 1.0 / jnp.maximum(h2, 1e-8)
        acc = jnp.zeros((16,), jnp.float32)
        for d in range(SIZE):
            v = plsc.load_gather(kbuf, [rows, jnp.full((16,), d, jnp.int32)])
            t = v * inv
            tb = plsc.bitcast(t, jnp.int32)
            tb = tb + 0x7FFF + (lax.shift_right_logical(tb, 16) & 1)
            t = plsc.bitcast(tb & jnp.int32(-65536), jnp.float32)
            acc = acc + t * pq[d]
        stage[...] = acc
        pltpu.sync_copy(stage, sims_out.at[pl.ds(j0 + h * 16, 16)])


def _scan_body(pqb_hbm, keys_hbm, sims_out, idx_out,
               pqb_v, buf0, buf1, kbuf, stage_s, stage_i,
               sem0, sem1, semg):
    cid = lax.axis_index("c")
    sid = lax.axis_index("s")
    wid = sid * NC + cid
    base_row = R_TC + wid * RPT

    pltpu.sync_copy(pqb_hbm, pqb_v)
    pqvecs = [pqb_v[pl.ds(k * 16, 16)] for k in range(SIZE // 16)]
    pq = [pqvecs[d // 16][d % 16] for d in range(SIZE)]
    riota = lax.iota(jnp.int32, 16)

    def start(cidx, buf, sem):
        off = base_row + cidx * CHUNK
        pltpu.async_copy(keys_hbm.at[pl.ds(off, CHUNK), :], buf, sem)

    def wait(buf, sem):
        pltpu.make_async_copy(keys_hbm.at[pl.ds(0, CHUNK), :], buf, sem).wait()

    def process(buf, chunk_row_base, bk, bi):
        def gbody(gr, carry):
            bk, bi = carry
            rows = riota + gr * 16
            zero = gr * 0
            dot = jnp.zeros((16,), jnp.float32)
            ss = jnp.zeros((16,), jnp.float32)
            for d in range(SIZE):
                cols = jnp.full((16,), zero + d, jnp.int32)
                v = plsc.load_gather(buf, [rows, cols])
                dot = dot + v * pq[d]
                ss = ss + v * v
            key = dot * jnp.abs(dot) / jnp.maximum(ss, 1e-16)
            idxv = riota + (chunk_row_base + gr * 16)
            take = key > bk
            bk = jnp.where(take, key, bk)
            bi = jnp.where(take, idxv, bi)
            return bk, bi
        return lax.fori_loop(0, GROUPS, gbody, (bk, bi))

    start(0, buf0, sem0)
    start(1, buf1, sem1)
    bk0 = jnp.full((16,), -jnp.inf, jnp.float32)
    bi0 = jnp.zeros((16,), jnp.int32)

    def cbody(g, carry):
        bk, bi = carry
        for b, (buf, sem) in enumerate(((buf0, sem0), (buf1, sem1))):
            cidx = 2 * g + b
            wait(buf, sem)
            bk, bi = process(buf, base_row + cidx * CHUNK, bk, bi)

            @pl.when(cidx + 2 < NCHUNK)
            def _():
                start(cidx + 2, buf, sem)
        return bk, bi

    _, bi = lax.fori_loop(0, NCHUNK // 2, cbody, (bk0, bi0))

    stage_i[...] = bi
    for j in range(16):
        pltpu.async_copy(keys_hbm.at[pl.ds(bi[j], 1), :],
                         kbuf.at[pl.ds(j, 1), :], semg)
    for j in range(16):
        pltpu.make_async_copy(keys_hbm.at[pl.ds(0, 1), :],
                              kbuf.at[pl.ds(j, 1), :], semg).wait()

    ss = jnp.zeros((16,), jnp.float32)
    for d in range(SIZE):
        v = plsc.load_gather(kbuf, [riota, jnp.full((16,), d, jnp.int32)])
        ss = ss + v * v
    ssc = jnp.maximum(ss, 1e-30)
    yi = jnp.int32(0x5F3759DF) - lax.shift_right_logical(
        plsc.bitcast(ssc, jnp.int32), 1)
    y = plsc.bitcast(yi, jnp.float32)
    for _ in range(3):
        y = y * (1.5 - 0.5 * ssc * y * y)
    h = ssc * y
    h = 0.5 * (h + ssc / h)
    inv = 1.0 / jnp.maximum(h, 1e-8)
    acc = jnp.zeros((16,), jnp.float32)
    for d in range(SIZE):
        v = plsc.load_gather(kbuf, [riota, jnp.full((16,), d, jnp.int32)])
        t = v * inv
        tb = plsc.bitcast(t, jnp.int32)
        tb = tb + 0x7FFF + (lax.shift_right_logical(tb, 16) & 1)
        t = plsc.bitcast(tb & jnp.int32(-65536), jnp.float32)
        acc = acc + t * pq[d]

    stage_s[...] = acc
    pltpu.sync_copy(stage_s, sims_out.at[pl.ds(wid * 16, 16)])
    pltpu.sync_copy(stage_i, idx_out.at[pl.ds(wid * 16, 16)])


def _pick_body(sims_hbm, idx_hbm, tsims_hbm, tidx_hbm, vals_hbm,
               conf_out, val_out,
               sbuf, ibuf, tsbuf, tibuf, cbuf, rowbuf, sem):
    cid = lax.axis_index("c")
    sid = lax.axis_index("s")
    wid = sid * NC + cid

    @pl.when(wid == 0)
    def _():
        pltpu.sync_copy(sims_hbm, sbuf)
        pltpu.sync_copy(idx_hbm, ibuf)
        pltpu.sync_copy(tsims_hbm, tsbuf)
        pltpu.sync_copy(tidx_hbm, tibuf)
        bs = sbuf[pl.ds(0, 16)]
        bi = ibuf[pl.ds(0, 16)]
        for t in range(1, NW):
            sv = sbuf[pl.ds(t * 16, 16)]
            iv = ibuf[pl.ds(t * 16, 16)]
            take = (sv > bs) | ((sv == bs) & (iv < bi))
            bs = jnp.where(take, sv, bs)
            bi = jnp.where(take, iv, bi)
        for t in range(NTC // 16):
            sv = tsbuf[pl.ds(t * 16, 16)]
            iv = tibuf[pl.ds(t * 16, 16)]
            take = (sv > bs) | ((sv == bs) & (iv < bi))
            bs = jnp.where(take, sv, bs)
            bi = jnp.where(take, iv, bi)
        mx = jnp.max(bs)
        cand = jnp.where(bs == mx, bi, jnp.int32(0x7FFFFFFF))
        bidx = jnp.min(cand)
        cbuf[...] = jnp.full((16,), mx, jnp.float32)
        pltpu.sync_copy(cbuf, conf_out)
        pltpu.async_copy(vals_hbm.at[pl.ds(bidx, 1), :], rowbuf, sem).wait()
        pltpu.sync_copy(rowbuf.at[0], val_out)


def _mesh():
    return plsc.VectorSubcoreMesh(core_axis_name="c", subcore_axis_name="s",
                                  num_cores=NC, num_subcores=NS)


def kernel(query, W, b, cache_keys, cache_values):
    pqn = _project(query, W, b)
    pqn_flat = pqn.reshape(SIZE)

    scan = pl.kernel(
        _scan_body,
        out_type=(jax.ShapeDtypeStruct((NFIN,), jnp.float32),
                  jax.ShapeDtypeStruct((NFIN,), jnp.int32)),
        mesh=_mesh(),
        scratch_types=[
            pltpu.VMEM((SIZE,), jnp.float32),
            pltpu.VMEM((CHUNK, SIZE), jnp.float32),
            pltpu.VMEM((CHUNK, SIZE), jnp.float32),
            pltpu.VMEM((16, SIZE), jnp.float32),
            pltpu.VMEM((16,), jnp.float32),
            pltpu.VMEM((16,), jnp.int32),
            pltpu.SemaphoreType.DMA,
            pltpu.SemaphoreType.DMA,
            pltpu.SemaphoreType.DMA,
        ],
        **_SC_PARAMS,
    )
    sims, fidx = scan(pqn_flat, cache_keys)
    tcidx = _tcscan(pqn, cache_keys).reshape(NTC)

    rescore = pl.kernel(
        _rescore_body,
        out_type=jax.ShapeDtypeStruct((NTC,), jnp.float32),
        mesh=_mesh(),
        scratch_types=[
            pltpu.VMEM((SIZE,), jnp.float32),
            pltpu.VMEM((NTC,), jnp.int32),
            pltpu.VMEM((TPW, SIZE), jnp.float32),
            pltpu.VMEM((16,), jnp.float32),
            pltpu.VMEM((16,), jnp.float32),
            pltpu.SemaphoreType.DMA,
        ],
        **_SC_PARAMS,
    )
    tsims = rescore(pqn_flat, tcidx, cache_keys)

    pick = pl.kernel(
        _pick_body,
        out_type=(jax.ShapeDtypeStruct((16,), jnp.float32),
                  jax.ShapeDtypeStruct((SIZE,), jnp.float32)),
        mesh=_mesh(),
        scratch_types=[
            pltpu.VMEM((NFIN,), jnp.float32),
            pltpu.VMEM((NFIN,), jnp.int32),
            pltpu.VMEM((NTC,), jnp.float32),
            pltpu.VMEM((NTC,), jnp.int32),
            pltpu.VMEM((16,), jnp.float32),
            pltpu.VMEM((1, SIZE), jnp.float32),
            pltpu.SemaphoreType.DMA,
        ],
        **_SC_PARAMS,
    )
    conf, row = pick(sims, fidx, tsims, tcidx, cache_values)
    return row.reshape(1, SIZE), conf[0]
